# baseline (speedup 1.0000x reference)
import functools

import jax
import jax.numpy as jnp
from jax import lax
from jax.experimental import pallas as pl
from jax.experimental.pallas import tpu as pltpu
from jax.experimental.pallas import tpu_sc as plsc

N_FIELDS = 26
VOCAB = 100000
EMBED_DIM = 32
BATCH = 1024
HIST = 20
LANES = 16
NVEC = BATCH // LANES
NST = 6

_mesh = plsc.VectorSubcoreMesh(core_axis_name="c", subcore_axis_name="s")


@functools.partial(
    pl.kernel,
    out_type=jax.ShapeDtypeStruct((N_FIELDS, HIST, EMBED_DIM, BATCH), jnp.float32),
    mesh=_mesh,
    compiler_params=pltpu.CompilerParams(needs_layout_passes=False),
    scratch_types=(
        [pltpu.VMEM((VOCAB,), jnp.float32),
         pltpu.VMEM((HIST, BATCH), jnp.int32)]
        + [pltpu.VMEM((BATCH,), jnp.float32) for _ in range(NST)]
        + [pltpu.SemaphoreType.DMA, pltpu.SemaphoreType.DMA]
        + [pltpu.SemaphoreType.DMA for _ in range(NST)]
    ),
)
def _lookup_t(idx_hbm, tab_hbm, out_hbm, *scr):
    row_v, idx_v = scr[0], scr[1]
    st_bufs = scr[2:2 + NST]
    sem_row, sem_idx = scr[2 + NST], scr[3 + NST]
    st_sems = scr[4 + NST:]

    w = lax.axis_index("s") * 2 + lax.axis_index("c")

    def drain_store(sb):
        pltpu.make_async_copy(tab_hbm.at[0, 0, pl.ds(0, BATCH)],
                              st_bufs[sb], st_sems[sb]).wait()

    def per_field(f, carry):
        pltpu.async_copy(tab_hbm.at[f, w], row_v, sem_row)
        pltpu.async_copy(idx_hbm.at[f], idx_v, sem_idx)
        pltpu.make_async_copy(idx_hbm.at[0], idx_v, sem_idx).wait()
        pltpu.make_async_copy(tab_hbm.at[0, 0], row_v, sem_row).wait()
        for h in range(HIST):
            sb = h % NST
            if h < NST:
                @pl.when(f > 0)
                def _():
                    drain_store(sb)
            else:
                drain_store(sb)

            st_buf = st_bufs[sb]

            @plsc.parallel_loop(0, NVEC, unroll=8)
            def _body(s):
                sl = pl.ds(s * LANES, LANES)
                st_buf[sl] = plsc.load_gather(row_v, [idx_v[h, sl]])

            pltpu.async_copy(st_buf, out_hbm.at[f, h, w], st_sems[sb])
        return carry

    lax.fori_loop(0, N_FIELDS, per_field, 0)
    for s in range(NST):
        drain_store(s)


def kernel(inputs, tables):
    tab_t = jnp.transpose(tables, (0, 2, 1))
    idx_t = jnp.transpose(inputs, (0, 2, 1))
    out_t = _lookup_t(idx_t, tab_t)
    return jnp.transpose(out_t, (0, 3, 1, 2))
